# baseline (speedup 1.0000x reference)
import functools

import jax
import jax.numpy as jnp
from jax.experimental import pallas as pl
from jax.experimental.pallas import tpu as pltpu

B, C, S, HID = 16, 12, 576, 768
SP1 = S + 1
PATCH_NUM = 84
H1DIM = 512


def _leaky(v):
    return jnp.where(v >= 0, v, 0.2 * v)


def _gcn_body(x_ref, hid_ref, w1_ref, w2_ref, hs_ref):
    def accum(c, acc):
        return acc + x_ref[0, c]

    xsum = jax.lax.fori_loop(1, C, accum, x_ref[0, 0])
    xm = xsum * (1.0 / C)
    rows = jax.lax.broadcasted_iota(jnp.int32, (SP1, SP1), 0)
    cols = jax.lax.broadcasted_iota(jnp.int32, (SP1, SP1), 1)
    pwf = jnp.where((rows > 0) & (cols > 0), xm, 0.0)

    def sc_accum(c, acc):
        return acc + x_ref[0, c, 0:1, :]

    score_sum = jax.lax.fori_loop(1, C, sc_accum, x_ref[0, 0, 0:1, :])
    meanrow = score_sum * (1.0 / C)

    mr = jnp.where(cols[0:1] > 0, meanrow, -jnp.inf)
    mx = jnp.max(mr)
    bi = jnp.min(jnp.where(mr == mx, cols[0:1], SP1))

    structure = jnp.transpose(x_ref[0, :, 0, :], (1, 0))
    support1 = jnp.dot(structure, w1_ref[...], preferred_element_type=jnp.float32)

    pw_bf = pwf.astype(jnp.bfloat16)
    h1 = jnp.dot(pw_bf, support1.astype(jnp.bfloat16),
                 preferred_element_type=jnp.float32)
    h1 = jnp.maximum(h1, 0.0)

    onehot = (cols[0:1] == bi).astype(jnp.bfloat16)
    pw_row = jnp.dot(onehot, pw_bf, preferred_element_type=jnp.float32)
    v = jnp.dot(pw_row.astype(jnp.bfloat16), h1.astype(jnp.bfloat16),
                preferred_element_type=jnp.float32)
    w = jnp.dot(v.astype(jnp.bfloat16), w2_ref[...].astype(jnp.bfloat16),
                preferred_element_type=jnp.float32)
    si_row = _leaky(w)

    hs_ref[0] = hid_ref[0]
    hs_ref[0, 0:1, :] = hid_ref[0, 0:1, :] + si_row


def _gcn_hs(x, hidden_states, W1, W2):
    return pl.pallas_call(
        _gcn_body,
        grid=(B,),
        in_specs=[
            pl.BlockSpec((1, C, SP1, SP1), lambda b: (b, 0, 0, 0)),
            pl.BlockSpec((1, SP1, HID), lambda b: (b, 0, 0)),
            pl.BlockSpec((C, H1DIM), lambda b: (0, 0)),
            pl.BlockSpec((H1DIM, HID), lambda b: (0, 0)),
        ],
        out_specs=pl.BlockSpec((1, SP1, HID), lambda b: (b, 0, 0)),
        out_shape=jax.ShapeDtypeStruct((B, SP1, HID), jnp.float32),
    )(x, hidden_states, W1, W2)


def _selection(score, hs):
    _, select_indices = jax.lax.top_k(score, PATCH_NUM)
    flat_idx = select_indices.reshape(B, -1)
    count = jnp.zeros((B, S), dtype=jnp.float32)
    count = count.at[jnp.arange(B)[:, None], flat_idx].add(1.0)
    k = jnp.array([[1.0, 2.0, 1.0], [2.0, 4.0, 2.0], [1.0, 2.0, 1.0]],
                  dtype=jnp.float32).reshape(1, 1, 3, 3)
    Himg = 24
    cimg = count.reshape(B, 1, Himg, Himg)
    count = jax.lax.conv_general_dilated(
        cimg, k, window_strides=(1, 1), padding='SAME',
        dimension_numbers=('NCHW', 'OIHW', 'NCHW')).reshape(B, -1)
    order = jnp.argsort(-count, axis=-1)
    patch_idx = order[:, :PATCH_NUM] + 1
    selected = hs[jnp.arange(B)[:, None], patch_idx]
    return patch_idx, selected


def kernel(hidden_states, x, contribution, W1, W2):
    hs = _gcn_hs(x, hidden_states, W1, W2)
    score = x[:, :, 0, 1:]
    patch_idx, selected = _selection(score, hs)
    return (hs, selected, patch_idx)
